# baseline (speedup 1.0000x reference)
import functools

import jax
import jax.numpy as jnp
from jax import lax
from jax.experimental import pallas as pl
from jax.experimental.pallas import tpu as pltpu
from jax.experimental.pallas import tpu_sc as plsc

N_NODES = 10000
D_FEAT = 128
N_EDGES = 320000
K_SIGN = 1000.0
EPSILON = 5.0

NC = 2
NS = 16
NW = NC * NS
ROW_LANES = 125
EROWS = N_EDGES // ROW_LANES
KROWS = EROWS // NW
N_TRASH = 240
N_ACC = N_NODES + N_TRASH
ZCH = N_ACC // NS // 16

ROWB = 10240
GRID_C = (N_ACC + ROWB - 1) // ROWB


def _proj_body(x_ref, w_ref, o_ref):
    o_ref[...] = jnp.dot(x_ref[...], w_ref[...],
                         preferred_element_type=jnp.float32)


def _project(x, W):
    return pl.pallas_call(
        _proj_body,
        out_shape=jax.ShapeDtypeStruct((N_NODES, 1), jnp.float32),
    )(x, W)


@functools.cache
def _make_sc_scatter():
    mesh = plsc.VectorSubcoreMesh(
        core_axis_name="c", subcore_axis_name="s",
        num_cores=NC, num_subcores=NS,
    )
    return functools.partial(
        pl.kernel,
        out_type=jax.ShapeDtypeStruct((NC, N_ACC), jnp.float32),
        mesh=mesh,
        scratch_types=[
            pltpu.VMEM((KROWS, ROW_LANES), jnp.int32),
            pltpu.VMEM((KROWS, ROW_LANES), jnp.int32),
            pltpu.VMEM((KROWS, ROW_LANES), jnp.float32),
            pltpu.VMEM((N_ACC // NS,), jnp.float32),
            pltpu.VMEM_SHARED((N_NODES,), jnp.float32),
            pltpu.VMEM_SHARED((N_ACC,), jnp.float32),
            pltpu.SemaphoreType.DMA,
            pltpu.SemaphoreType.DMA,
        ],
    )(_sc_scatter_body)


def _sc_scatter_body(xw_hbm, edge_hbm, out_hbm,
                     isrc, idst, val, zbuf, table_sh, acc_sh, gsem, ssem):
    c = lax.axis_index("c")
    s = lax.axis_index("s")
    wid = c * NS + s
    per = N_ACC // NS
    base = KROWS * wid

    ld0 = pltpu.async_copy(edge_hbm.at[0, pl.ds(base, KROWS)], isrc, ssem)
    ld1 = pltpu.async_copy(edge_hbm.at[1, pl.ds(base, KROWS)], idst, ssem)

    def _zfill(i, carry):
        zbuf[pl.ds(i * 16, 16)] = jnp.zeros((16,), jnp.float32)
        return carry
    lax.fori_loop(0, ZCH, _zfill, 0)
    pltpu.sync_copy(zbuf, acc_sh.at[pl.ds(s * per, per)])

    @pl.when(s == 0)
    def _stage_table():
        pltpu.sync_copy(xw_hbm, table_sh)

    plsc.subcore_barrier()
    ld0.wait()
    ld1.wait()

    def _fire_gather(g, carry):
        pltpu.async_copy(table_sh.at[isrc.at[g]], val.at[g], gsem)
        return carry
    lax.fori_loop(0, KROWS, _fire_gather, 0)

    def _drain_gather(g, carry):
        pltpu.make_async_copy(table_sh.at[isrc.at[g]], val.at[g], gsem).wait()
        return carry
    lax.fori_loop(0, KROWS, _drain_gather, 0)

    def _fire_scat(g, carry):
        pltpu.async_copy(val.at[g], acc_sh.at[idst.at[g]], ssem, add=True)
        return carry
    lax.fori_loop(0, KROWS, _fire_scat, 0)

    def _drain_scat(g, carry):
        pltpu.make_async_copy(val.at[g], acc_sh.at[idst.at[g]], ssem).wait()
        return carry
    lax.fori_loop(0, KROWS, _drain_scat, 0)

    plsc.subcore_barrier()

    @pl.when(s == 0)
    def _flush():
        pltpu.sync_copy(acc_sh, out_hbm.at[c])


def _pair_body(part_ref, o_ref):
    p = part_ref[...]
    xsum = (p[0:1, :] + p[1:2, :]) * K_SIGN
    ids = lax.broadcasted_iota(jnp.int32, (1, N_ACC), 1)
    colk = jnp.where(ids < N_NODES, xsum, 1e33)
    i = pl.program_id(0)
    seg = (part_ref[0:1, pl.ds(i * ROWB, ROWB)]
           + part_ref[1:2, pl.ds(i * ROWB, ROWB)]) * K_SIGN
    rowk = jnp.reshape(seg, (ROWB, 1)) - EPSILON
    acc = jnp.sum(jnp.tanh(rowk - colk), axis=1, keepdims=True)
    o_ref[...] = acc + jnp.float32(N_TRASH)


def _pairwise(part):
    return pl.pallas_call(
        _pair_body,
        grid=(GRID_C,),
        in_specs=[
            pl.BlockSpec((NC, N_ACC), lambda i: (0, 0)),
        ],
        out_specs=pl.BlockSpec((ROWB, 1), lambda i: (i, 0)),
        out_shape=jax.ShapeDtypeStruct((N_NODES, 1), jnp.float32),
        compiler_params=pltpu.CompilerParams(
            dimension_semantics=("arbitrary",),
        ),
    )(part)


def kernel(x, edge_index, W):
    edge3 = edge_index.astype(jnp.int32).reshape(2, EROWS, ROW_LANES)
    xw = _project(x, W).reshape(N_NODES)
    part = _make_sc_scatter()(xw, edge3)
    out = _pairwise(part)
    return out.reshape(N_NODES)


# speedup vs baseline: 1.0183x; 1.0183x over previous
import functools

import jax
import jax.numpy as jnp
from jax import lax
from jax.experimental import pallas as pl
from jax.experimental.pallas import tpu as pltpu
from jax.experimental.pallas import tpu_sc as plsc

N_NODES = 10000
D_FEAT = 128
N_EDGES = 320000
K_SIGN = 1000.0
EPSILON = 5.0

NC = 2
NS = 16
NW = NC * NS
ROW_LANES = 125
EROWS = N_EDGES // ROW_LANES
KROWS = EROWS // NW
N_TRASH = 240
N_ACC = N_NODES + N_TRASH
ZCH = N_ACC // NS // 16

ROWB = 2048
GRID_C = (N_ACC + ROWB - 1) // ROWB


def _proj_body(x_ref, w_ref, o_ref):
    o_ref[...] = jnp.dot(x_ref[...], w_ref[...],
                         preferred_element_type=jnp.float32)


def _project(x, W):
    return pl.pallas_call(
        _proj_body,
        out_shape=jax.ShapeDtypeStruct((N_NODES, 1), jnp.float32),
    )(x, W)


@functools.cache
def _make_sc_scatter():
    mesh = plsc.VectorSubcoreMesh(
        core_axis_name="c", subcore_axis_name="s",
        num_cores=NC, num_subcores=NS,
    )
    return functools.partial(
        pl.kernel,
        out_type=jax.ShapeDtypeStruct((NC, N_ACC), jnp.float32),
        mesh=mesh,
        scratch_types=[
            pltpu.VMEM((KROWS, ROW_LANES), jnp.int32),
            pltpu.VMEM((KROWS, ROW_LANES), jnp.int32),
            pltpu.VMEM((KROWS, ROW_LANES), jnp.float32),
            pltpu.VMEM((N_ACC // NS,), jnp.float32),
            pltpu.VMEM_SHARED((N_NODES,), jnp.float32),
            pltpu.VMEM_SHARED((N_ACC,), jnp.float32),
            pltpu.SemaphoreType.DMA,
            pltpu.SemaphoreType.DMA,
        ],
    )(_sc_scatter_body)


def _sc_scatter_body(xw_hbm, edge_hbm, out_hbm,
                     isrc, idst, val, zbuf, table_sh, acc_sh, gsem, ssem):
    c = lax.axis_index("c")
    s = lax.axis_index("s")
    wid = c * NS + s
    per = N_ACC // NS
    base = KROWS * wid

    ld0 = pltpu.async_copy(edge_hbm.at[0, pl.ds(base, KROWS)], isrc, ssem)
    ld1 = pltpu.async_copy(edge_hbm.at[1, pl.ds(base, KROWS)], idst, ssem)

    def _zfill(i, carry):
        zbuf[pl.ds(i * 16, 16)] = jnp.zeros((16,), jnp.float32)
        return carry
    lax.fori_loop(0, ZCH, _zfill, 0)
    pltpu.sync_copy(zbuf, acc_sh.at[pl.ds(s * per, per)])

    @pl.when(s == 0)
    def _stage_table():
        pltpu.sync_copy(xw_hbm, table_sh)

    plsc.subcore_barrier()
    ld0.wait()
    ld1.wait()

    UNR = 4
    def _fire_gather(g, carry):
        for j in range(UNR):
            pltpu.async_copy(table_sh.at[isrc.at[g * UNR + j]],
                             val.at[g * UNR + j], gsem)
        return carry
    lax.fori_loop(0, KROWS // UNR, _fire_gather, 0)

    def _drain_gather(g, carry):
        for j in range(UNR):
            pltpu.make_async_copy(table_sh.at[isrc.at[g * UNR + j]],
                                  val.at[g * UNR + j], gsem).wait()
        return carry
    lax.fori_loop(0, KROWS // UNR, _drain_gather, 0)

    def _fire_scat(g, carry):
        for j in range(UNR):
            pltpu.async_copy(val.at[g * UNR + j],
                             acc_sh.at[idst.at[g * UNR + j]], ssem, add=True)
        return carry
    lax.fori_loop(0, KROWS // UNR, _fire_scat, 0)

    def _drain_scat(g, carry):
        for j in range(UNR):
            pltpu.make_async_copy(val.at[g * UNR + j],
                                  acc_sh.at[idst.at[g * UNR + j]],
                                  ssem).wait()
        return carry
    lax.fori_loop(0, KROWS // UNR, _drain_scat, 0)

    plsc.subcore_barrier()

    @pl.when(s == 0)
    def _flush():
        pltpu.sync_copy(acc_sh, out_hbm.at[c])


def _pair_body(part_ref, o_ref):
    p = part_ref[...]
    xsum = (p[0:1, :] + p[1:2, :]) * K_SIGN
    ids = lax.broadcasted_iota(jnp.int32, (1, N_ACC), 1)
    colk = jnp.where(ids < N_NODES, xsum, 1e33)
    i = pl.program_id(0)
    seg = (part_ref[0:1, pl.ds(i * ROWB, ROWB)]
           + part_ref[1:2, pl.ds(i * ROWB, ROWB)]) * K_SIGN
    rowk = jnp.reshape(seg, (ROWB, 1)) - EPSILON
    acc = jnp.sum(jnp.tanh(rowk - colk), axis=1, keepdims=True)
    o_ref[...] = acc + jnp.float32(N_TRASH)


def _pairwise(part):
    return pl.pallas_call(
        _pair_body,
        grid=(GRID_C,),
        in_specs=[
            pl.BlockSpec((NC, N_ACC), lambda i: (0, 0)),
        ],
        out_specs=pl.BlockSpec((ROWB, 1), lambda i: (i, 0)),
        out_shape=jax.ShapeDtypeStruct((N_NODES, 1), jnp.float32),
        compiler_params=pltpu.CompilerParams(
            dimension_semantics=("arbitrary",),
        ),
    )(part)


def kernel(x, edge_index, W):
    edge3 = edge_index.astype(jnp.int32).reshape(2, EROWS, ROW_LANES)
    xw = _project(x, W).reshape(N_NODES)
    part = _make_sc_scatter()(xw, edge3)
    out = _pairwise(part)
    return out.reshape(N_NODES)


# speedup vs baseline: 1.0957x; 1.0760x over previous
import functools

import jax
import jax.numpy as jnp
from jax import lax
from jax.experimental import pallas as pl
from jax.experimental.pallas import tpu as pltpu
from jax.experimental.pallas import tpu_sc as plsc

N_NODES = 10000
D_FEAT = 128
N_EDGES = 320000
K_SIGN = 1000.0
EPSILON = 5.0

NC = 2
NS = 16
NW = NC * NS
ROW_LANES = 125
EROWS = N_EDGES // ROW_LANES
KROWS = EROWS // NW
N_TRASH = 240
N_ACC = N_NODES + N_TRASH
ZCH = N_ACC // NS // 16

ROWB = 2048
GRID_C = (N_ACC + ROWB - 1) // ROWB


def _proj_body(x_ref, w_ref, o_ref):
    o_ref[...] = lax.dot_general(
        w_ref[...], x_ref[...], (((0,), (1,)), ((), ())),
        preferred_element_type=jnp.float32)


def _project(x, W):
    return pl.pallas_call(
        _proj_body,
        out_shape=jax.ShapeDtypeStruct((1, N_NODES), jnp.float32),
    )(x, W)


@functools.cache
def _make_sc_scatter():
    mesh = plsc.VectorSubcoreMesh(
        core_axis_name="c", subcore_axis_name="s",
        num_cores=NC, num_subcores=NS,
    )
    return functools.partial(
        pl.kernel,
        out_type=jax.ShapeDtypeStruct((NC, N_ACC), jnp.float32),
        mesh=mesh,
        scratch_types=[
            pltpu.VMEM((KROWS, ROW_LANES), jnp.int32),
            pltpu.VMEM((KROWS, ROW_LANES), jnp.int32),
            pltpu.VMEM((KROWS, ROW_LANES), jnp.float32),
            pltpu.VMEM((N_ACC // NS,), jnp.float32),
            pltpu.VMEM_SHARED((N_NODES,), jnp.float32),
            pltpu.VMEM_SHARED((N_ACC,), jnp.float32),
            pltpu.SemaphoreType.DMA,
            pltpu.SemaphoreType.DMA,
        ],
    )(_sc_scatter_body)


def _sc_scatter_body(xw_hbm, edge_hbm, out_hbm,
                     isrc, idst, val, zbuf, table_sh, acc_sh, gsem, ssem):
    c = lax.axis_index("c")
    s = lax.axis_index("s")
    wid = c * NS + s
    per = N_ACC // NS
    base = KROWS * wid

    ld0 = pltpu.async_copy(edge_hbm.at[0, pl.ds(base, KROWS)], isrc, ssem)
    ld1 = pltpu.async_copy(edge_hbm.at[1, pl.ds(base, KROWS)], idst, ssem)

    def _zfill(i, carry):
        zbuf[pl.ds(i * 16, 16)] = jnp.zeros((16,), jnp.float32)
        return carry
    lax.fori_loop(0, ZCH, _zfill, 0)
    pltpu.sync_copy(zbuf, acc_sh.at[pl.ds(s * per, per)])

    @pl.when(s == 0)
    def _stage_table():
        pltpu.sync_copy(xw_hbm.at[0], table_sh)

    plsc.subcore_barrier()
    ld0.wait()
    ld1.wait()

    UNR = 4
    def _fire_gather(g, carry):
        for j in range(UNR):
            pltpu.async_copy(table_sh.at[isrc.at[g * UNR + j]],
                             val.at[g * UNR + j], gsem)
        return carry
    lax.fori_loop(0, KROWS // UNR, _fire_gather, 0)

    def _drain_gather(g, carry):
        for j in range(UNR):
            pltpu.make_async_copy(table_sh.at[isrc.at[g * UNR + j]],
                                  val.at[g * UNR + j], gsem).wait()
        return carry
    lax.fori_loop(0, KROWS // UNR, _drain_gather, 0)

    def _fire_scat(g, carry):
        for j in range(UNR):
            pltpu.async_copy(val.at[g * UNR + j],
                             acc_sh.at[idst.at[g * UNR + j]], ssem, add=True)
        return carry
    lax.fori_loop(0, KROWS // UNR, _fire_scat, 0)

    def _drain_scat(g, carry):
        for j in range(UNR):
            pltpu.make_async_copy(val.at[g * UNR + j],
                                  acc_sh.at[idst.at[g * UNR + j]],
                                  ssem).wait()
        return carry
    lax.fori_loop(0, KROWS // UNR, _drain_scat, 0)

    plsc.subcore_barrier()

    @pl.when(s == 0)
    def _flush():
        pltpu.sync_copy(acc_sh, out_hbm.at[c])


def _pair_body(part_ref, o_ref):
    p = part_ref[...]
    xsum = (p[0:1, :] + p[1:2, :]) * K_SIGN
    ids = lax.broadcasted_iota(jnp.int32, (1, N_ACC), 1)
    colk = jnp.where(ids < N_NODES, xsum, 1e33)
    i = pl.program_id(0)
    seg = (part_ref[0:1, pl.ds(i * ROWB, ROWB)]
           + part_ref[1:2, pl.ds(i * ROWB, ROWB)]) * K_SIGN
    rowk = jnp.reshape(seg, (ROWB, 1)) - EPSILON
    acc = jnp.sum(jnp.tanh(rowk - colk), axis=1, keepdims=True)
    o_ref[...] = jnp.reshape(acc, (1, ROWB)) + jnp.float32(N_TRASH)


def _pairwise(part):
    return pl.pallas_call(
        _pair_body,
        grid=(GRID_C,),
        in_specs=[
            pl.BlockSpec((NC, N_ACC), lambda i: (0, 0)),
        ],
        out_specs=pl.BlockSpec((1, ROWB), lambda i: (0, i)),
        out_shape=jax.ShapeDtypeStruct((1, N_NODES), jnp.float32),
        compiler_params=pltpu.CompilerParams(
            dimension_semantics=("arbitrary",),
        ),
    )(part)


def kernel(x, edge_index, W):
    edge3 = edge_index.astype(jnp.int32).reshape(2, EROWS, ROW_LANES)
    xw = _project(x, W)
    part = _make_sc_scatter()(xw, edge3)
    out = _pairwise(part)
    return out.reshape(N_NODES)
